# Initial kernel scaffold; baseline (speedup 1.0000x reference)
import jax
import jax.numpy as jnp
from jax.experimental import pallas as pl


def kernel(reward, probs, not_done):
    raise NotImplementedError("write your pallas kernel here")


# speedup vs baseline: 62.8308x; 62.8308x over previous
import functools

import jax
import jax.numpy as jnp
from jax import lax
from jax.experimental import pallas as pl
from jax.experimental.pallas import tpu as pltpu
from jax.experimental.pallas import tpu_sc as plsc

V_MIN = -10.0
V_MAX = 10.0
N_ATOMS = 51
DISCOUNT = 0.99
DELTA = (V_MAX - V_MIN) / (N_ATOMS - 1)
INV_DELTA = 1.0 / DELTA

try:
    _info = plsc.get_sparse_core_info()
    NC, NS = _info.num_cores, _info.num_subcores
except Exception:
    NC, NS = 2, 16
NW = NC * NS


def _body(rows_per, block, reward_h, probs_h, ndone_h, out_h,
          inbuf, acc, rbuf, ndbuf):
    A = N_ATOMS
    nb = rows_per // block
    wid = lax.axis_index("s") * NC + lax.axis_index("c")
    lane = lax.iota(jnp.int32, 16)
    zeros16 = jnp.zeros((16,), jnp.float32)

    def block_body(blk, carry):
        row0 = wid * rows_per + blk * block
        pltpu.sync_copy(probs_h.at[pl.ds(row0 * A, block * A)], inbuf)
        pltpu.sync_copy(reward_h.at[pl.ds(row0, block)], rbuf)
        pltpu.sync_copy(ndone_h.at[pl.ds(row0, block)], ndbuf)

        def zero_body(z, c):
            acc[pl.ds(z * 16, 16)] = zeros16
            return c

        lax.fori_loop(0, (block * A) // 16, zero_body, 0, unroll=8)

        def group_body(g, c):
            s = g * 16
            rvec = rbuf[pl.ds(s, 16)]
            cvec = DISCOUNT * ndbuf[pl.ds(s, 16)]
            rowbase = (s + lane) * A
            rowtop = rowbase + (A - 1)

            def j_body(j, cc):
                jsplat = jnp.full((16,), j, jnp.int32)
                atomv = V_MIN + DELTA * jsplat.astype(jnp.float32)
                nav = rvec + cvec * atomv
                nav = jnp.minimum(jnp.maximum(nav, V_MIN), V_MAX)
                b = (nav - V_MIN) * INV_DELTA
                li = b.astype(jnp.int32)
                frac = b - li.astype(jnp.float32)
                p = plsc.load_gather(inbuf, [rowbase + jsplat])
                wl = (1.0 - frac) * p
                wu = frac * p
                idx_l = rowbase + li
                idx_u = jnp.minimum(idx_l + 1, rowtop)
                plsc.addupdate_scatter(acc, [idx_l], wl)
                plsc.addupdate_scatter(acc, [idx_u], wu)
                return cc

            lax.fori_loop(0, A, j_body, 0)
            return c

        lax.fori_loop(0, block // 16, group_body, 0)
        pltpu.sync_copy(acc, out_h.at[pl.ds(row0 * A, block * A)])
        return carry

    lax.fori_loop(0, nb, block_body, 0)


def kernel(reward, probs, not_done):
    bs, A = probs.shape
    assert A == N_ATOMS
    rows_per = bs // NW
    block = 1024
    mesh = plsc.VectorSubcoreMesh(
        core_axis_name="c", subcore_axis_name="s",
        num_cores=NC, num_subcores=NS)
    run = functools.partial(
        pl.kernel,
        out_type=jax.ShapeDtypeStruct((bs * A,), jnp.float32),
        mesh=mesh,
        compiler_params=pltpu.CompilerParams(needs_layout_passes=False),
        scratch_types=[
            pltpu.VMEM((block * A,), jnp.float32),
            pltpu.VMEM((block * A,), jnp.float32),
            pltpu.VMEM((block,), jnp.float32),
            pltpu.VMEM((block,), jnp.float32),
        ],
    )(functools.partial(_body, rows_per, block))
    out = run(reward.reshape(-1), probs.reshape(-1), not_done.reshape(-1))
    return out.reshape(bs, A)
